# baseline (speedup 1.0000x reference)
import functools

import jax
import jax.numpy as jnp
from jax import lax
from jax.experimental import pallas as pl
from jax.experimental.pallas import tpu as pltpu
from jax.experimental.pallas import tpu_sc as plsc

K_NN = 16
_INTERPRET = False


def _gelu(x):
    u = 0.7978845608028654 * (x + 0.044715 * (x * x * x))
    t = 2.0 / (1.0 + jnp.exp(-2.0 * u)) - 1.0
    return 0.5 * x * (1.0 + t)


def _prep_body(N, NP, Co, xf_ref, w_ref, b_ref, xn_ref, a_ref, c_ref):
    xf = xf_ref[0]
    ss = jnp.sum(xf * xf, axis=0, keepdims=True)
    xn = xf / jnp.maximum(jnp.sqrt(ss), 1e-12)
    xn_ref[0, :, :N] = xn
    if NP > N:
        xn_ref[0, :, N:] = jnp.zeros((xf.shape[0], NP - N), jnp.float32)
    ac = lax.dot_general(xf, w_ref[...], (((0,), (0,)), ((), ())),
                         preferred_element_type=jnp.float32,
                         precision=lax.Precision.HIGHEST)
    a_ref[0] = ac[:, :Co] + b_ref[...]
    c_ref[0, :, :Co] = ac[:, Co:]
    c_ref[0, :, Co:] = jnp.zeros((ac.shape[0], 128 - Co), jnp.float32)


def _prep(xf, wcat, b2, NP):
    B, C, N = xf.shape
    Co = wcat.shape[1] // 2
    return pl.pallas_call(
        functools.partial(_prep_body, N, NP, Co),
        grid=(B,),
        in_specs=[
            pl.BlockSpec((1, C, N), lambda i: (i, 0, 0)),
            pl.BlockSpec((C, 2 * Co), lambda i: (0, 0)),
            pl.BlockSpec((1, Co), lambda i: (0, 0)),
        ],
        out_specs=[
            pl.BlockSpec((1, C, NP), lambda i: (i, 0, 0)),
            pl.BlockSpec((1, N, Co), lambda i: (i, 0, 0)),
            pl.BlockSpec((1, N, 128), lambda i: (i, 0, 0)),
        ],
        out_shape=[
            jax.ShapeDtypeStruct((B, C, NP), jnp.float32),
            jax.ShapeDtypeStruct((B, N, Co), jnp.float32),
            jax.ShapeDtypeStruct((B, N, 128), jnp.float32),
        ],
        interpret=_INTERPRET,
    )(xf, wcat, b2)


def _topk_body(N, NP, TQ, xa_ref, xq_ref, nnt_ref, gidxt_ref, d_ref):
    xa = xa_ref[0]
    xq = xq_ref[0]
    dot = lax.dot_general(xa.astype(jnp.bfloat16), xq.astype(jnp.bfloat16),
                          (((0,), (0,)), ((), ())),
                          preferred_element_type=jnp.float32)
    sq_a = jnp.sum(xa * xa, axis=0)[:, None]
    sq_q = jnp.sum(xq * xq, axis=0)[None, :]
    d = (sq_q + (-2.0 * dot)) + sq_a
    row = lax.broadcasted_iota(jnp.int32, (NP, TQ), 0)
    if NP > N:
        d = jnp.where(row >= N, jnp.inf, d)
    d_ref[...] = d
    big = jnp.int32(2 ** 30)
    boff = pl.program_id(0) * N

    def _round(r, carry):
        m_prev, am_prev = carry
        dcur = d_ref[...]
        keep = (dcur > m_prev) | ((dcur == m_prev) & (row > am_prev))
        cand = jnp.where(keep, dcur, jnp.inf)
        m = jnp.min(cand, axis=0, keepdims=True)
        am = jnp.min(jnp.where(cand == m, row, big), axis=0, keepdims=True)
        nnt_ref[0, pl.ds(r, 1), :] = am
        gidxt_ref[0, pl.ds(r, 1), :] = am + boff
        return (m, am)

    m0 = jnp.full((1, d.shape[1]), -jnp.inf, jnp.float32)
    am0 = jnp.full((1, d.shape[1]), -1, jnp.int32)
    lax.fori_loop(0, K_NN, _round, (m0, am0))


def _topk(xn, N):
    B, C, NP = xn.shape
    TQ = NP
    for t in (640, 512, 384, 256, 128):
        if NP % t == 0:
            TQ = t
            break
    return pl.pallas_call(
        functools.partial(_topk_body, N, NP, TQ),
        grid=(B, NP // TQ),
        in_specs=[
            pl.BlockSpec((1, C, NP), lambda i, j: (i, 0, 0)),
            pl.BlockSpec((1, C, TQ), lambda i, j: (i, 0, j)),
        ],
        out_specs=[
            pl.BlockSpec((1, K_NN, TQ), lambda i, j: (i, 0, j)),
            pl.BlockSpec((1, K_NN, TQ), lambda i, j: (i, 0, j)),
        ],
        out_shape=[
            jax.ShapeDtypeStruct((B, K_NN, NP), jnp.int32),
            jax.ShapeDtypeStruct((B, K_NN, NP), jnp.int32),
        ],
        scratch_shapes=[pltpu.VMEM((NP, TQ), jnp.float32)],
        interpret=_INTERPRET,
    )(xn, xn)


def _edgeconv_sc(a2, c2, gidx):
    NQ, Co = a2.shape
    Cg = c2.shape[1]
    NW = 32
    assert NQ % NW == 0 and Co % 16 == 0
    PW = NQ // NW
    QB = 8
    while PW % QB:
        QB //= 2
    NB = PW // QB
    KROW = QB * K_NN
    NC6 = Co // 16
    gidx3 = gidx.reshape(NW, NB, KROW)
    mesh = plsc.VectorSubcoreMesh(core_axis_name="c", subcore_axis_name="s")

    @functools.partial(
        pl.kernel, mesh=mesh,
        out_type=jax.ShapeDtypeStruct((NQ, Co), jnp.float32),
        scratch_types=[
            pltpu.VMEM((NB, KROW), jnp.int32),
            pltpu.VMEM((PW, Co), jnp.float32),
            pltpu.VMEM((PW, Co), jnp.float32),
            pltpu.VMEM((KROW, Cg), jnp.float32),
            pltpu.SemaphoreType.DMA,
        ],
    )
    def body(a2_hbm, c2_hbm, gidx_hbm, out_hbm, idx_v, a_v, out_v, rows_v, sem):
        wid = lax.axis_index("s") * 2 + lax.axis_index("c")
        base = wid * PW
        pltpu.sync_copy(gidx_hbm.at[wid], idx_v)
        pltpu.sync_copy(a2_hbm.at[pl.ds(base, PW)], a_v)

        def batch_body(g, carry):
            pltpu.async_copy(c2_hbm.at[idx_v.at[g]], rows_v, sem).wait()
            for qq in range(QB):
                q = g * QB + qq
                rbase = qq * K_NN
                first = tuple(rows_v[rbase, pl.ds(c * 16, 16)]
                              for c in range(NC6))

                def k_body(k, mm):
                    mns, mxs = mm
                    rows = [rows_v[rbase + k, pl.ds(c * 16, 16)]
                            for c in range(NC6)]
                    return (tuple(jnp.minimum(a, r) for a, r in zip(mns, rows)),
                            tuple(jnp.maximum(a, r) for a, r in zip(mxs, rows)))

                mns, mxs = lax.fori_loop(1, K_NN, k_body, (first, first))
                for c in range(NC6):
                    av = a_v[q, pl.ds(c * 16, 16)]
                    res = jnp.maximum(_gelu(av + mns[c]), _gelu(av + mxs[c]))
                    out_v[q, pl.ds(c * 16, 16)] = res
            return carry

        lax.fori_loop(0, NB, batch_body, 0)
        pltpu.sync_copy(out_v, out_hbm.at[pl.ds(base, PW)])

    return body(a2, c2, gidx3)


def kernel(x, W, b):
    B, C, H, Wd = x.shape
    N = H * Wd
    Co = W.shape[1]
    NP = -(-N // 128) * 128
    xf = x.reshape(B, C, N)
    wcat = jnp.concatenate([W[:C] - W[C:], W[C:]], axis=1)
    b2 = b.reshape(1, Co)

    xn, a2, c2 = _prep(xf, wcat, b2, NP)
    nn_full, gidx_full = _topk(xn, N)

    nn = nn_full[:, :, :N].transpose(0, 2, 1)
    gidx = gidx_full[:, :, :N].transpose(0, 2, 1).reshape(B * N, K_NN)

    out_rows = _edgeconv_sc(a2.reshape(B * N, Co), c2.reshape(B * N, 128),
                            gidx)
    out = out_rows.reshape(B, N, Co).transpose(0, 2, 1).reshape(B, Co, H, Wd)

    center = jnp.broadcast_to(
        jnp.arange(N, dtype=nn.dtype)[None, :, None], (B, N, K_NN))
    edge_index = jnp.stack([nn, center], axis=0)
    return (out, edge_index)


# speedup vs baseline: 1.4105x; 1.4105x over previous
import functools

import jax
import jax.numpy as jnp
from jax import lax
from jax.experimental import pallas as pl
from jax.experimental.pallas import tpu as pltpu
from jax.experimental.pallas import tpu_sc as plsc

K_NN = 16


def _gelu(x):
    u = 0.7978845608028654 * (x + 0.044715 * (x * x * x))
    t = 2.0 / (1.0 + jnp.exp(-2.0 * u)) - 1.0
    return 0.5 * x * (1.0 + t)


def _prep_body(N, NP, Co, xf_ref, w_ref, b_ref, xn_ref, a_ref, c_ref):
    xf = xf_ref[0]
    ss = jnp.sum(xf * xf, axis=0, keepdims=True)
    xn = xf / jnp.maximum(jnp.sqrt(ss), 1e-12)
    xn_ref[0, :, :N] = xn
    if NP > N:
        xn_ref[0, :, N:] = jnp.zeros((xf.shape[0], NP - N), jnp.float32)
    ac = lax.dot_general(xf, w_ref[...], (((0,), (0,)), ((), ())),
                         preferred_element_type=jnp.float32,
                         precision=lax.Precision.HIGHEST)
    a_ref[0] = ac[:, :Co] + b_ref[...]
    c_ref[0, :, :Co] = ac[:, Co:]
    c_ref[0, :, Co:] = jnp.zeros((ac.shape[0], 128 - Co), jnp.float32)


def _prep(xf, wcat, b2, NP):
    B, C, N = xf.shape
    Co = wcat.shape[1] // 2
    return pl.pallas_call(
        functools.partial(_prep_body, N, NP, Co),
        grid=(B,),
        in_specs=[
            pl.BlockSpec((1, C, N), lambda i: (i, 0, 0)),
            pl.BlockSpec((C, 2 * Co), lambda i: (0, 0)),
            pl.BlockSpec((1, Co), lambda i: (0, 0)),
        ],
        out_specs=[
            pl.BlockSpec((1, C, NP), lambda i: (i, 0, 0)),
            pl.BlockSpec((1, N, Co), lambda i: (i, 0, 0)),
            pl.BlockSpec((1, N, 128), lambda i: (i, 0, 0)),
        ],
        out_shape=[
            jax.ShapeDtypeStruct((B, C, NP), jnp.float32),
            jax.ShapeDtypeStruct((B, N, Co), jnp.float32),
            jax.ShapeDtypeStruct((B, N, 128), jnp.float32),
        ],
    )(xf, wcat, b2)


def _topk_body(N, NP, TQ, xa_ref, xq_ref, nnt_ref, gidxt_ref, d_ref):
    xa = xa_ref[0]
    xq = xq_ref[0]
    dot = lax.dot_general(xa.astype(jnp.bfloat16), xq.astype(jnp.bfloat16),
                          (((0,), (0,)), ((), ())),
                          preferred_element_type=jnp.float32)
    sq_a = jnp.sum(xa * xa, axis=0)[:, None]
    sq_q = jnp.sum(xq * xq, axis=0)[None, :]
    d = (sq_q + (-2.0 * dot)) + sq_a
    row = lax.broadcasted_iota(jnp.int32, (NP, TQ), 0)
    if NP > N:
        d = jnp.where(row >= N, jnp.inf, d)
    d_ref[...] = d
    big = jnp.int32(2 ** 30)
    boff = pl.program_id(0) * N

    def _round(r, carry):
        dcur = d_ref[...]
        m = jnp.min(dcur, axis=0, keepdims=True)
        am = jnp.min(jnp.where(dcur == m, row, big), axis=0, keepdims=True)
        nnt_ref[0, pl.ds(r, 1), :] = am
        gidxt_ref[0, pl.ds(r, 1), :] = am + boff
        d_ref[...] = jnp.where(row == am, jnp.inf, dcur)
        return carry

    lax.fori_loop(0, K_NN, _round, 0)


def _topk(xn, N):
    B, C, NP = xn.shape
    TQ = NP
    for t in (640, 512, 384, 256, 128):
        if NP % t == 0:
            TQ = t
            break
    return pl.pallas_call(
        functools.partial(_topk_body, N, NP, TQ),
        grid=(B, NP // TQ),
        in_specs=[
            pl.BlockSpec((1, C, NP), lambda i, j: (i, 0, 0)),
            pl.BlockSpec((1, C, TQ), lambda i, j: (i, 0, j)),
        ],
        out_specs=[
            pl.BlockSpec((1, K_NN, TQ), lambda i, j: (i, 0, j)),
            pl.BlockSpec((1, K_NN, TQ), lambda i, j: (i, 0, j)),
        ],
        out_shape=[
            jax.ShapeDtypeStruct((B, K_NN, NP), jnp.int32),
            jax.ShapeDtypeStruct((B, K_NN, NP), jnp.int32),
        ],
        scratch_shapes=[pltpu.VMEM((NP, TQ), jnp.float32)],
    )(xn, xn)


def _edgeconv_sc(a2, c2, gidx):
    NQ, Co = a2.shape
    Cg = c2.shape[1]
    NW = 32
    assert NQ % NW == 0 and Co % 16 == 0
    PW = NQ // NW
    QB = 8
    while PW % QB:
        QB //= 2
    NB = PW // QB
    KROW = QB * K_NN
    NC6 = Co // 16
    gidx3 = gidx.reshape(NW, NB, KROW)
    mesh = plsc.VectorSubcoreMesh(core_axis_name="c", subcore_axis_name="s")

    @functools.partial(
        pl.kernel, mesh=mesh,
        out_type=jax.ShapeDtypeStruct((NQ, Co), jnp.float32),
        scratch_types=[
            pltpu.VMEM((NB, KROW), jnp.int32),
            pltpu.VMEM((PW, Co), jnp.float32),
            pltpu.VMEM((PW, Co), jnp.float32),
            pltpu.VMEM((KROW, Cg), jnp.float32),
            pltpu.SemaphoreType.DMA,
        ],
    )
    def body(a2_hbm, c2_hbm, gidx_hbm, out_hbm, idx_v, a_v, out_v, rows_v, sem):
        wid = lax.axis_index("s") * 2 + lax.axis_index("c")
        base = wid * PW
        pltpu.sync_copy(gidx_hbm.at[wid], idx_v)
        pltpu.sync_copy(a2_hbm.at[pl.ds(base, PW)], a_v)

        def batch_body(g, carry):
            pltpu.async_copy(c2_hbm.at[idx_v.at[g]], rows_v, sem).wait()
            for qq in range(QB):
                q = g * QB + qq
                rbase = qq * K_NN
                first = tuple(rows_v[rbase, pl.ds(c * 16, 16)]
                              for c in range(NC6))

                def k_body(k, mm):
                    mns, mxs = mm
                    rows = [rows_v[rbase + k, pl.ds(c * 16, 16)]
                            for c in range(NC6)]
                    return (tuple(jnp.minimum(a, r) for a, r in zip(mns, rows)),
                            tuple(jnp.maximum(a, r) for a, r in zip(mxs, rows)))

                mns, mxs = lax.fori_loop(1, K_NN, k_body, (first, first))
                for c in range(NC6):
                    av = a_v[q, pl.ds(c * 16, 16)]
                    res = jnp.maximum(_gelu(av + mns[c]), _gelu(av + mxs[c]))
                    out_v[q, pl.ds(c * 16, 16)] = res
            return carry

        lax.fori_loop(0, NB, batch_body, 0)
        pltpu.sync_copy(out_v, out_hbm.at[pl.ds(base, PW)])

    return body(a2, c2, gidx3)


def kernel(x, W, b):
    B, C, H, Wd = x.shape
    N = H * Wd
    Co = W.shape[1]
    NP = -(-N // 128) * 128
    xf = x.reshape(B, C, N)
    wcat = jnp.concatenate([W[:C] - W[C:], W[C:]], axis=1)
    b2 = b.reshape(1, Co)

    xn, a2, c2 = _prep(xf, wcat, b2, NP)
    nn_full, gidx_full = _topk(xn, N)

    nn = nn_full[:, :, :N].transpose(0, 2, 1)
    gidx = gidx_full[:, :, :N].transpose(0, 2, 1).reshape(B * N, K_NN)

    out_rows = _edgeconv_sc(a2.reshape(B * N, Co), c2.reshape(B * N, 128),
                            gidx)
    out = out_rows.reshape(B, N, Co).transpose(0, 2, 1).reshape(B, Co, H, Wd)

    center = jnp.broadcast_to(
        jnp.arange(N, dtype=nn.dtype)[None, :, None], (B, N, K_NN))
    edge_index = jnp.stack([nn, center], axis=0)
    return (out, edge_index)
